# baseline (speedup 1.0000x reference)
import jax
import jax.numpy as jnp
from jax.experimental import pallas as pl
from jax.experimental.pallas import tpu as pltpu


def _copy_body(u_ref, i_ref, u_out, i_out, sem_u, sem_i):
    cu = pltpu.make_async_copy(u_ref, u_out, sem_u)
    ci = pltpu.make_async_copy(i_ref, i_out, sem_i)
    cu.start()
    ci.start()
    cu.wait()
    ci.wait()


def kernel(embedding_user, embedding_item):
    return pl.pallas_call(
        _copy_body,
        in_specs=[
            pl.BlockSpec(memory_space=pl.ANY),
            pl.BlockSpec(memory_space=pl.ANY),
        ],
        out_specs=[
            pl.BlockSpec(memory_space=pl.ANY),
            pl.BlockSpec(memory_space=pl.ANY),
        ],
        out_shape=[
            jax.ShapeDtypeStruct(embedding_user.shape, embedding_user.dtype),
            jax.ShapeDtypeStruct(embedding_item.shape, embedding_item.dtype),
        ],
        scratch_shapes=[pltpu.SemaphoreType.DMA, pltpu.SemaphoreType.DMA],
    )(embedding_user, embedding_item)
